# baseline (speedup 1.0000x reference)
import functools

import jax
import jax.numpy as jnp
from jax.experimental import pallas as pl
from jax.experimental.pallas import tpu as pltpu

_K = 16
_RADIUS2 = 0.0625
_BISECT_STEPS = 14
_ROW_BLOCK = 128
_COL_CHUNK = 512


def _knn_loss_block(pc_blk_ref, pcT_ref, flow_blk_ref, flowT_ref, out_ref,
                    scr_ref):
    b = pl.program_id(0)
    i = pl.program_id(1)

    pc_blk = pc_blk_ref[0]
    pcT = pcT_ref[0]
    flow_blk = flow_blk_ref[0]

    rb, n_cols = scr_ref.shape
    kf = jnp.float32(_K)

    pb = pc_blk.astype(jnp.bfloat16).astype(jnp.float32)
    pt = pcT.astype(jnp.bfloat16).astype(jnp.float32)
    dot = (pb[:, 0:1] * pt[0:1, :]
           + pb[:, 1:2] * pt[1:2, :]
           + pb[:, 2:3] * pt[2:3, :])
    sq_r = jnp.sum(pc_blk * pc_blk, axis=1, keepdims=True)
    sq_c = jnp.sum(pcT * pcT, axis=0, keepdims=True)
    d2 = jnp.maximum(sq_r + sq_c - 2.0 * dot, 0.0)
    scr_ref[...] = d2

    c_hi0 = jnp.sum((d2 <= _RADIUS2).astype(jnp.float32), axis=1,
                    keepdims=True)
    rowmin = jnp.min(d2, axis=1, keepdims=True)
    lo0 = jnp.full_like(c_hi0, -1.0)
    hi0 = jnp.full_like(c_hi0, _RADIUS2)
    c_lo0 = jnp.zeros_like(c_hi0)

    def body(_, st):
        lo, hi, c_lo, c_hi = st
        mid = 0.5 * (lo + hi)
        cnt = jnp.sum((scr_ref[...] <= mid).astype(jnp.float32), axis=1,
                      keepdims=True)
        pred = cnt >= kf
        lo_n = jnp.where(pred, lo, mid)
        c_lo_n = jnp.where(pred, c_lo, cnt)
        hi_n = jnp.where(pred, mid, hi)
        c_hi_n = jnp.where(pred, cnt, c_hi)
        return lo_n, hi_n, c_lo_n, c_hi_n

    lo, hi, c_lo, c_hi = jax.lax.fori_loop(
        0, _BISECT_STEPS, body, (lo0, hi0, c_lo0, c_hi0))

    fx = flow_blk[:, 0:1]
    fy = flow_blk[:, 1:2]
    fz = flow_blk[:, 2:3]
    big = jnp.int32(n_cols)

    def fin(c, st):
        s_lo, s_hi, am, l1m = st
        sl = pl.ds(c * _COL_CHUNK, _COL_CHUNK)
        d2c = scr_ref[:, sl]
        ft = flowT_ref[0, :, sl]
        l1c = (jnp.abs(fx - ft[0:1, :])
               + jnp.abs(fy - ft[1:2, :])
               + jnp.abs(fz - ft[2:3, :]))
        s_lo = s_lo + jnp.sum(jnp.where(d2c <= lo, l1c, 0.0), axis=1,
                              keepdims=True)
        s_hi = s_hi + jnp.sum(jnp.where(d2c <= hi, l1c, 0.0), axis=1,
                              keepdims=True)
        iota = (jax.lax.broadcasted_iota(jnp.int32, (rb, _COL_CHUNK), 1)
                + c * _COL_CHUNK)
        cand = jnp.where(d2c == rowmin, iota, big)
        amc = jnp.min(cand, axis=1, keepdims=True)
        l1_at = jnp.sum(jnp.where(cand == amc, l1c, 0.0), axis=1,
                        keepdims=True)
        take = amc < am
        l1m = jnp.where(take, l1_at, l1m)
        am = jnp.minimum(am, amc)
        return s_lo, s_hi, am, l1m

    z = jnp.zeros_like(c_hi0)
    s_lo, s_hi, _, l1min = jax.lax.fori_loop(
        0, n_cols // _COL_CHUNK, fin,
        (z, z, jnp.full((rb, 1), big, jnp.int32), z))

    denom = jnp.maximum(c_hi - c_lo, 1.0)
    sel = jnp.where(c_hi <= kf,
                    s_hi,
                    s_lo + (kf - c_lo) * (s_hi - s_lo) / denom)
    sel = sel + jnp.maximum(kf - c_hi0, 0.0) * l1min

    part = jnp.sum(sel).reshape(1, 1)

    @pl.when(jnp.logical_and(b == 0, i == 0))
    def _init():
        out_ref[...] = jnp.zeros_like(out_ref)

    out_ref[...] += part


def kernel(pc, flow):
    B, N, _ = pc.shape
    rb = _ROW_BLOCK
    pcT = jnp.transpose(pc, (0, 2, 1))
    flowT = jnp.transpose(flow, (0, 2, 1))

    grid = (B, N // rb)
    total = pl.pallas_call(
        _knn_loss_block,
        grid=grid,
        in_specs=[
            pl.BlockSpec((1, rb, 3), lambda b, i: (b, i, 0)),
            pl.BlockSpec((1, 3, N), lambda b, i: (b, 0, 0)),
            pl.BlockSpec((1, rb, 3), lambda b, i: (b, i, 0)),
            pl.BlockSpec((1, 3, N), lambda b, i: (b, 0, 0)),
        ],
        out_specs=pl.BlockSpec((1, 1), lambda b, i: (0, 0)),
        out_shape=jax.ShapeDtypeStruct((1, 1), jnp.float32),
        scratch_shapes=[pltpu.VMEM((rb, N), jnp.float32)],
    )(pc, pcT, flow, flowT)

    return total[0, 0] / jnp.float32(B * N * _K)
